# baseline (device time: 285591 ns/iter reference)
import jax
import jax.numpy as jnp
from jax import lax
from jax.experimental import pallas as pl
from jax.experimental.pallas import tpu as pltpu

T_FIX = 64


def kernel(x, A, B, C):
    b, s, d = x.shape
    n = A.shape[1]

    def body(x_ref, a_ref, b_ref, c_ref, out_ref, state_ref, hinit_ref,
             send_sem, recv_sem):
        my_x = lax.axis_index("x")
        my_y = lax.axis_index("y")
        partner = (1 - my_x, my_y)

        barrier = pltpu.get_barrier_semaphore()
        pl.semaphore_signal(barrier, inc=1, device_id=partner,
                            device_id_type=pl.DeviceIdType.MESH)
        pl.semaphore_wait(barrier, 1)

        dA = jnp.exp(a_ref[...]).T

        def step(t, h):
            xt = x_ref[:, t, :]
            bt = b_ref[:, t, :]
            ct = c_ref[:, t, :]
            h = h * dA[None] + bt[:, :, None] * xt[:, None, :]
            out_ref[:, t, :] = jnp.sum(h * ct[:, :, None], axis=1)
            return h

        h_final = lax.fori_loop(0, s, step, jnp.zeros((b, n, d), jnp.float32))
        state_ref[...] = h_final

        rdma = pltpu.make_async_remote_copy(
            src_ref=state_ref, dst_ref=hinit_ref,
            send_sem=send_sem, recv_sem=recv_sem,
            device_id=partner, device_id_type=pl.DeviceIdType.MESH)

        @pl.when(my_x == 0)
        def _():
            rdma.start()
            rdma.wait_send()

        @pl.when(my_x == 1)
        def _():
            rdma.wait_recv()
            lax.fori_loop(0, T_FIX, step, hinit_ref[...])

    out_shape = jax.ShapeDtypeStruct((b, s, d), jnp.float32)
    return pl.pallas_call(
        body,
        out_shape=out_shape,
        in_specs=[pl.BlockSpec(memory_space=pltpu.VMEM)] * 4,
        out_specs=pl.BlockSpec(memory_space=pltpu.VMEM),
        scratch_shapes=[
            pltpu.VMEM((b, n, d), jnp.float32),
            pltpu.VMEM((b, n, d), jnp.float32),
            pltpu.SemaphoreType.DMA,
            pltpu.SemaphoreType.DMA,
        ],
        compiler_params=pltpu.CompilerParams(collective_id=0),
    )(x, A, B, C)
